# Initial kernel scaffold; baseline (speedup 1.0000x reference)
import jax
import jax.numpy as jnp
from jax.experimental import pallas as pl


def kernel(vertices, segments, segment_edges, segment_mask, segment_edges_mask, params):
    raise NotImplementedError("write your pallas kernel here")


# speedup vs baseline: 3.7852x; 3.7852x over previous
import numpy as np
import jax
import jax.numpy as jnp
from jax import lax
from jax.experimental import pallas as pl
from jax.experimental.pallas import tpu as pltpu
from jax.experimental.pallas import tpu_sc as plsc

_NC, _NS, _CH = 2, 16, 128
_NW = _NC * _NS


def _sc_mesh():
    return plsc.VectorSubcoreMesh(core_axis_name="c", subcore_axis_name="s")


def _sc_gather(table, idx):
    V, W = table.shape
    N = idx.shape[0]
    nch = N // (_NW * _CH)
    idx4 = idx.reshape(_NC, _NS, nch, _CH)

    def body(table_hbm, idx_hbm, out_hbm, idx_v, rows_v, sem):
        c = lax.axis_index("c")
        s = lax.axis_index("s")
        for k in range(nch):
            pltpu.sync_copy(idx_hbm.at[c, s, k], idx_v)
            pltpu.async_copy(table_hbm.at[idx_v], rows_v, sem).wait()
            base = ((c * _NS + s) * nch + k) * _CH
            pltpu.sync_copy(rows_v, out_hbm.at[pl.ds(base, _CH)])

    fn = pl.kernel(
        body,
        out_type=jax.ShapeDtypeStruct((N, W), jnp.float32),
        mesh=_sc_mesh(),
        compiler_params=pltpu.CompilerParams(use_tc_tiling_on_sc=False),
        scratch_types=[
            pltpu.VMEM((_CH,), jnp.int32),
            pltpu.VMEM((_CH, W), jnp.float32),
            pltpu.SemaphoreType.DMA,
        ],
    )
    return fn(table, idx4)


def _sc_segsum(vals, gidx, sidx, nbins):
    V, W = vals.shape
    N = gidx.shape[0]
    nch = N // (_NW * _CH)
    g4 = gidx.reshape(_NC, _NS, nch, _CH)
    s4 = sidx.reshape(_NC, _NS, nch, _CH)
    rows_per = nbins // _NS

    def body(vals_hbm, g_hbm, s_hbm, zero_hbm, out_hbm, gi_v, si_v, rows_v, acc, sem):
        c = lax.axis_index("c")
        s = lax.axis_index("s")
        r0 = s * rows_per
        pltpu.sync_copy(zero_hbm.at[pl.ds(r0, rows_per)], acc.at[pl.ds(r0, rows_per)])
        plsc.subcore_barrier()
        for k in range(nch):
            pltpu.sync_copy(g_hbm.at[c, s, k], gi_v)
            pltpu.async_copy(vals_hbm.at[gi_v], rows_v, sem).wait()
            pltpu.sync_copy(s_hbm.at[c, s, k], si_v)
            pltpu.sync_copy(rows_v, acc.at[si_v], add=True)
        plsc.subcore_barrier()
        pltpu.sync_copy(acc.at[pl.ds(r0, rows_per)], out_hbm.at[c, pl.ds(r0, rows_per)])

    zero = jnp.zeros((nbins, W), jnp.float32)
    fn = pl.kernel(
        body,
        out_type=jax.ShapeDtypeStruct((_NC, nbins, W), jnp.float32),
        mesh=_sc_mesh(),
        compiler_params=pltpu.CompilerParams(use_tc_tiling_on_sc=False),
        scratch_types=[
            pltpu.VMEM((_CH,), jnp.int32),
            pltpu.VMEM((_CH,), jnp.int32),
            pltpu.VMEM((_CH, W), jnp.float32),
            pltpu.VMEM_SHARED((nbins, W), jnp.float32),
            pltpu.SemaphoreType.DMA,
        ],
    )
    return fn(vals, g4, s4, zero)


def _tc_angle(ep0, ep1, angle_tab, cos_th):
    Ne = ep0.shape[0]

    def body(e0_ref, e1_ref, tab_ref, th_ref, out_ref):
        def dirn(e):
            d = e[:, 3:6] - e[:, 0:3]
            n = jnp.sqrt(jnp.sum(d * d, axis=1, keepdims=True) + 1e-12)
            return d / jnp.clip(n, 1e-8, None)

        c = jnp.abs(jnp.sum(dirn(e0_ref[...]) * dirn(e1_ref[...]), axis=1, keepdims=True))
        c = jnp.clip(c, 0.0, 1.0 - 1e-6)
        idx = jnp.sum((c <= th_ref[...]).astype(jnp.int32), axis=1)
        oh = (idx[:, None] == lax.broadcasted_iota(jnp.int32, (Ne, 128), 1)).astype(jnp.float32)
        vals = jnp.dot(oh, tab_ref[...], preferred_element_type=jnp.float32)
        out_ref[...] = jnp.concatenate([vals, jnp.ones((Ne, 16), jnp.float32)], axis=1)

    return pl.pallas_call(
        body, out_shape=jax.ShapeDtypeStruct((Ne, 32), jnp.float32)
    )(ep0, ep1, angle_tab, cos_th)


def _tc_embed(segrows, angA, angB, ct, lt, mt, dt, win, bin_, wp, bp):
    N = segrows.shape[0]

    def body(sr_ref, aA_ref, aB_ref, ct_ref, lt_ref, mt_ref, dt_ref,
             win_ref, bin_ref, wp_ref, bp_ref, x_out, xp_out):
        sr = sr_ref[...]
        v0 = sr[:, 0:3]
        v1 = sr[:, 3:6]
        midps = 0.5 * (v0 + v1)
        vec = v1 - v0
        length = jnp.sqrt(jnp.sum(vec * vec, axis=1, keepdims=True) + 1e-12)
        dirs = vec / jnp.clip(length, 1e-8, None)

        def disc(t, lo, hi):
            t = (t - lo) / (hi - lo) * 128.0 - 0.5
            return jnp.clip(jnp.round(t), 0, 127).astype(jnp.int32)

        def oh(ix):
            return (ix[:, None] == lax.broadcasted_iota(jnp.int32, (N, 128), 1)
                    ).astype(jnp.float32)

        def mm(a, b):
            return jnp.dot(a, b, preferred_element_type=jnp.float32)

        win = win_ref[...]
        x = jnp.broadcast_to(bin_ref[...][None, :], (N, 192))
        ixc = disc(jnp.concatenate([v0, v1], axis=1), -1.0, 1.0)
        ct_t = ct_ref[...]
        for k in range(6):
            x = x + mm(oh(ixc[:, k]), mm(ct_t, win[64 * k:64 * (k + 1), :]))
        ixl = disc(length, 0.0, 2.0)[:, 0]
        x = x + mm(oh(ixl), mm(lt_ref[...], win[384:400, :]))
        ixd = disc(dirs, -1.0, 1.0)
        dt_t = dt_ref[...]
        for k in range(3):
            x = x + mm(oh(ixd[:, k]), mm(dt_t, win[400 + 64 * k:400 + 64 * (k + 1), :]))
        aa = aA_ref[...] + aB_ref[...]
        amean = aa[:, 0:16] / jnp.clip(aa[:, 16:17], 1.0, None)
        x = x + mm(amean, win[592:608, :])
        ixm = disc(midps, -1.0, 1.0)
        mt_t = mt_ref[...]
        for k in range(3):
            x = x + mm(oh(ixm[:, k]), mm(mt_t, win[608 + 16 * k:608 + 16 * (k + 1), :]))
        x_out[...] = x
        xp_out[...] = jax.nn.relu(mm(x, wp_ref[...]) + bp_ref[...][None, :])

    return pl.pallas_call(
        body,
        out_shape=[
            jax.ShapeDtypeStruct((N, 192), jnp.float32),
            jax.ShapeDtypeStruct((N, wp.shape[1]), jnp.float32),
        ],
    )(segrows, angA, angB, ct, lt, mt, dt, win, bin_, wp, bp)


def _tc_sage(x, pA, pB, cA, cB, Wl, bl, Wr, br, ln=None, nxt=None):
    N, _ = x.shape
    o = Wl.shape[1]
    outs = [jax.ShapeDtypeStruct((N, o), jnp.float32)]
    ins = [x, pA, pB, cA, cB, Wl, bl, Wr, br]
    if ln is not None:
        ins += [ln[0], ln[1]]
    if nxt is not None:
        ins += [nxt[0], nxt[1]]
        outs.append(jax.ShapeDtypeStruct((N, nxt[0].shape[1]), jnp.float32))

    def body(*refs):
        it = iter(refs)
        x_r, pA_r, pB_r, cA_r, cB_r, Wl_r, bl_r, Wr_r, br_r = (next(it) for _ in range(9))
        if ln is not None:
            g_r, b_r = next(it), next(it)
        if nxt is not None:
            wp_r, bp_r = next(it), next(it)
        h_out = next(it)
        xp_out = next(it) if nxt is not None else None

        def mm(a, b):
            return jnp.dot(a, b, preferred_element_type=jnp.float32)

        cnt = jnp.clip((cA_r[...] + cB_r[...])[:, 0:1], 1.0, None)
        mean = (pA_r[...] + pB_r[...]) / cnt
        out = mm(mean, Wl_r[...]) + bl_r[...][None, :] + mm(x_r[...], Wr_r[...]) + br_r[...][None, :]
        nrm = jnp.sqrt(jnp.sum(out * out, axis=1, keepdims=True))
        h = out / jnp.clip(nrm, 1e-12, None)
        if ln is not None:
            h = h * jax.nn.sigmoid(h)
            mu = jnp.mean(h, axis=1, keepdims=True)
            var = jnp.mean((h - mu) ** 2, axis=1, keepdims=True)
            h = (h - mu) / jnp.sqrt(var + 1e-5) * g_r[...][None, :] + b_r[...][None, :]
        h_out[...] = h
        if nxt is not None:
            xp_out[...] = jax.nn.relu(mm(h, wp_r[...]) + bp_r[...][None, :])

    res = pl.pallas_call(body, out_shape=outs)(*ins)
    return res if nxt is not None else (res[0], None)


_COS_TH = np.concatenate(
    [np.cos(np.arange(1, 128, dtype=np.float64) * np.pi / 256.0), [-2.0]]
).astype(np.float32).reshape(1, 128)


def kernel(vertices, segments, segment_edges, segment_mask, segment_edges_mask, params):
    B, NV, _ = vertices.shape
    NL = segments.shape[1]
    E = segment_edges.shape[1]
    N = B * NL
    NE = 2 * B * E

    seg = jnp.where(segment_mask[..., None], segments, 0)
    se0 = jnp.where(segment_edges_mask[..., None], segment_edges, 0)

    vpad = jnp.pad(vertices.reshape(B * NV, 3), ((0, 0), (0, 13)))
    boffV = (jnp.arange(B) * NV)[:, None]
    iA = (seg[..., 0] + boffV).reshape(-1).astype(jnp.int32)
    iB = (seg[..., 1] + boffV).reshape(-1).astype(jnp.int32)
    g1 = _sc_gather(vpad, jnp.concatenate([iA, iB]))
    vAB = jnp.concatenate([g1[:N, :3], g1[N:, :3]], axis=1)
    segrows = jnp.pad(vAB, ((0, 0), (0, 10)))

    boffL = (jnp.arange(B) * NL)[:, None]
    e0 = (se0[..., 0] + boffL).reshape(-1).astype(jnp.int32)
    e1 = (se0[..., 1] + boffL).reshape(-1).astype(jnp.int32)
    g2 = _sc_gather(segrows, jnp.concatenate([e0, e1]))
    ep0, ep1 = g2[:B * E], g2[B * E:]

    av = _tc_angle(ep0, ep1, params['angle_tab'], jnp.asarray(_COS_TH))
    nbins_a = 4224
    boffA = (jnp.arange(B) * (NL + 1))[:, None]
    b0 = (se0[..., 0] + boffA).reshape(-1).astype(jnp.int32)
    b1 = (se0[..., 1] + boffA).reshape(-1).astype(jnp.int32)
    ea = jnp.arange(B * E, dtype=jnp.int32)
    pa = _sc_segsum(av, jnp.concatenate([ea, ea]), jnp.concatenate([b0, b1]), nbins_a)
    ar = pa[:, :B * (NL + 1)].reshape(_NC, B, NL + 1, 32)[:, :, :NL].reshape(_NC, N, 32)

    p = params
    x, xp = _tc_embed(segrows, ar[0], ar[1], p['coor_tab'], p['length_tab'],
                      p['midp_tab'], p['dir_tab'], p['Win'], p['bin'],
                      p['sage'][0]['Wp'], p['sage'][0]['bp'])

    src = jnp.concatenate([e0, e1])
    dst = jnp.concatenate([e1, e0])
    ones16 = jnp.ones((8, 16), jnp.float32)
    cp = _sc_segsum(ones16, jnp.zeros((NE,), jnp.int32), dst, N)
    h = x
    nlayers = len(p['sage'])
    for li in range(nlayers):
        pl_parts = _sc_segsum(xp, src, dst, N)
        sp = p['sage'][li]
        ln = (p['ln_g'], p['ln_b']) if li == 0 else None
        nxt = None
        if li + 1 < nlayers:
            nxt = (p['sage'][li + 1]['Wp'], p['sage'][li + 1]['bp'])
        h, xp = _tc_sage(h, pl_parts[0], pl_parts[1], cp[0], cp[1],
                         sp['Wl'], sp['bl'], sp['Wr'], sp['br'], ln=ln, nxt=nxt)
    return h.reshape(B, NL, 384)


# speedup vs baseline: 4.2096x; 1.1121x over previous
import numpy as np
import jax
import jax.numpy as jnp
from jax import lax
from jax.experimental import pallas as pl
from jax.experimental.pallas import tpu as pltpu
from jax.experimental.pallas import tpu_sc as plsc

_NC, _NS, _CH = 2, 16, 128
_NW = _NC * _NS


def _sc_mesh():
    return plsc.VectorSubcoreMesh(core_axis_name="c", subcore_axis_name="s")


def _sc_gather(table, idx):
    V, W = table.shape
    N = idx.shape[0]
    nch = N // (_NW * _CH)
    idx4 = idx.reshape(_NC, _NS, nch, _CH)

    def body(table_hbm, idx_hbm, out_hbm, idx_v, rows_v, sem0, sem1):
        c = lax.axis_index("c")
        s = lax.axis_index("s")
        sems = (sem0, sem1)
        pltpu.sync_copy(idx_hbm.at[c, s], idx_v)
        cps = [None, None]
        cps[0] = pltpu.async_copy(table_hbm.at[idx_v.at[0]], rows_v.at[0], sems[0])
        for k in range(nch):
            slot = k % 2
            if k + 1 < nch:
                nslot = (k + 1) % 2
                cps[nslot] = pltpu.async_copy(
                    table_hbm.at[idx_v.at[k + 1]], rows_v.at[nslot], sems[nslot])
            cps[slot].wait()
            base = ((c * _NS + s) * nch + k) * _CH
            pltpu.sync_copy(rows_v.at[slot], out_hbm.at[pl.ds(base, _CH)])

    fn = pl.kernel(
        body,
        out_type=jax.ShapeDtypeStruct((N, W), jnp.float32),
        mesh=_sc_mesh(),
        compiler_params=pltpu.CompilerParams(use_tc_tiling_on_sc=False),
        scratch_types=[
            pltpu.VMEM((nch, _CH), jnp.int32),
            pltpu.VMEM((2, _CH, W), jnp.float32),
            pltpu.SemaphoreType.DMA,
            pltpu.SemaphoreType.DMA,
        ],
    )
    return fn(table, idx4)


def _sc_segsum(vals, gidx, sidx, nbins):
    V, W = vals.shape
    N = gidx.shape[0]
    ch = 64 if W >= 256 else _CH
    nch = N // (_NW * ch)
    g4 = gidx.reshape(_NC, _NS, nch, ch)
    s4 = sidx.reshape(_NC, _NS, nch, ch)
    rows_per = nbins // _NS

    def body(vals_hbm, g_hbm, s_hbm, zero_hbm, out_hbm, gi_v, si_v, rows_v, acc, sem0, sem1):
        c = lax.axis_index("c")
        s = lax.axis_index("s")
        sems = (sem0, sem1)
        r0 = s * rows_per
        pltpu.sync_copy(g_hbm.at[c, s], gi_v)
        pltpu.sync_copy(s_hbm.at[c, s], si_v)
        cps = [None, None]
        cps[0] = pltpu.async_copy(vals_hbm.at[gi_v.at[0]], rows_v.at[0], sems[0])
        pltpu.sync_copy(zero_hbm.at[pl.ds(r0, rows_per)], acc.at[pl.ds(r0, rows_per)])
        plsc.subcore_barrier()
        for k in range(nch):
            slot = k % 2
            if k + 1 < nch:
                nslot = (k + 1) % 2
                cps[nslot] = pltpu.async_copy(
                    vals_hbm.at[gi_v.at[k + 1]], rows_v.at[nslot], sems[nslot])
            cps[slot].wait()
            pltpu.sync_copy(rows_v.at[slot], acc.at[si_v.at[k]], add=True)
        plsc.subcore_barrier()
        pltpu.sync_copy(acc.at[pl.ds(r0, rows_per)], out_hbm.at[c, pl.ds(r0, rows_per)])

    zero = jnp.zeros((nbins, W), jnp.float32)
    fn = pl.kernel(
        body,
        out_type=jax.ShapeDtypeStruct((_NC, nbins, W), jnp.float32),
        mesh=_sc_mesh(),
        compiler_params=pltpu.CompilerParams(use_tc_tiling_on_sc=False),
        scratch_types=[
            pltpu.VMEM((nch, ch), jnp.int32),
            pltpu.VMEM((nch, ch), jnp.int32),
            pltpu.VMEM((2, ch, W), jnp.float32),
            pltpu.VMEM_SHARED((nbins, W), jnp.float32),
            pltpu.SemaphoreType.DMA,
            pltpu.SemaphoreType.DMA,
        ],
    )
    return fn(vals, g4, s4, zero)


def _tc_angle(ep0, ep1, angle_tab, cos_th):
    Ne = ep0.shape[0]

    def body(e0_ref, e1_ref, tab_ref, th_ref, out_ref):
        def dirn(e):
            d = e[:, 3:6] - e[:, 0:3]
            n = jnp.sqrt(jnp.sum(d * d, axis=1, keepdims=True) + 1e-12)
            return d / jnp.clip(n, 1e-8, None)

        c = jnp.abs(jnp.sum(dirn(e0_ref[...]) * dirn(e1_ref[...]), axis=1, keepdims=True))
        c = jnp.clip(c, 0.0, 1.0 - 1e-6)
        idx = jnp.sum((c <= th_ref[...]).astype(jnp.int32), axis=1)
        oh = (idx[:, None] == lax.broadcasted_iota(jnp.int32, (Ne, 128), 1)).astype(jnp.float32)
        vals = jnp.dot(oh, tab_ref[...], preferred_element_type=jnp.float32)
        out_ref[...] = jnp.concatenate([vals, jnp.ones((Ne, 16), jnp.float32)], axis=1)

    return pl.pallas_call(
        body, out_shape=jax.ShapeDtypeStruct((Ne, 32), jnp.float32)
    )(ep0, ep1, angle_tab, cos_th)


def _tc_embed(segrows, angA, angB, ct, lt, mt, dt, win, bin_, wp, bp):
    N = segrows.shape[0]

    def body(sr_ref, aA_ref, aB_ref, ct_ref, lt_ref, mt_ref, dt_ref,
             win_ref, bin_ref, wp_ref, bp_ref, x_out, xp_out):
        sr = sr_ref[...]
        v0 = sr[:, 0:3]
        v1 = sr[:, 3:6]
        midps = 0.5 * (v0 + v1)
        vec = v1 - v0
        length = jnp.sqrt(jnp.sum(vec * vec, axis=1, keepdims=True) + 1e-12)
        dirs = vec / jnp.clip(length, 1e-8, None)

        def disc(t, lo, hi):
            t = (t - lo) / (hi - lo) * 128.0 - 0.5
            return jnp.clip(jnp.round(t), 0, 127).astype(jnp.int32)

        def oh(ix):
            return (ix[:, None] == lax.broadcasted_iota(jnp.int32, (N, 128), 1)
                    ).astype(jnp.float32)

        def mm(a, b):
            return jnp.dot(a, b, preferred_element_type=jnp.float32)

        win = win_ref[...]
        x = jnp.broadcast_to(bin_ref[...][None, :], (N, 192))
        ixc = disc(jnp.concatenate([v0, v1], axis=1), -1.0, 1.0)
        ct_t = ct_ref[...]
        for k in range(6):
            x = x + mm(oh(ixc[:, k]), mm(ct_t, win[64 * k:64 * (k + 1), :]))
        ixl = disc(length, 0.0, 2.0)[:, 0]
        x = x + mm(oh(ixl), mm(lt_ref[...], win[384:400, :]))
        ixd = disc(dirs, -1.0, 1.0)
        dt_t = dt_ref[...]
        for k in range(3):
            x = x + mm(oh(ixd[:, k]), mm(dt_t, win[400 + 64 * k:400 + 64 * (k + 1), :]))
        aa = aA_ref[...] + aB_ref[...]
        amean = aa[:, 0:16] / jnp.clip(aa[:, 16:17], 1.0, None)
        x = x + mm(amean, win[592:608, :])
        ixm = disc(midps, -1.0, 1.0)
        mt_t = mt_ref[...]
        for k in range(3):
            x = x + mm(oh(ixm[:, k]), mm(mt_t, win[608 + 16 * k:608 + 16 * (k + 1), :]))
        x_out[...] = x
        xp_out[...] = jax.nn.relu(mm(x, wp_ref[...]) + bp_ref[...][None, :])

    return pl.pallas_call(
        body,
        out_shape=[
            jax.ShapeDtypeStruct((N, 192), jnp.float32),
            jax.ShapeDtypeStruct((N, wp.shape[1]), jnp.float32),
        ],
    )(segrows, angA, angB, ct, lt, mt, dt, win, bin_, wp, bp)


def _tc_sage(x, pA, pB, cA, cB, Wl, bl, Wr, br, ln=None, nxt=None):
    N, _ = x.shape
    o = Wl.shape[1]
    outs = [jax.ShapeDtypeStruct((N, o), jnp.float32)]
    ins = [x, pA, pB, cA, cB, Wl, bl, Wr, br]
    if ln is not None:
        ins += [ln[0], ln[1]]
    if nxt is not None:
        ins += [nxt[0], nxt[1]]
        outs.append(jax.ShapeDtypeStruct((N, nxt[0].shape[1]), jnp.float32))

    def body(*refs):
        it = iter(refs)
        x_r, pA_r, pB_r, cA_r, cB_r, Wl_r, bl_r, Wr_r, br_r = (next(it) for _ in range(9))
        if ln is not None:
            g_r, b_r = next(it), next(it)
        if nxt is not None:
            wp_r, bp_r = next(it), next(it)
        h_out = next(it)
        xp_out = next(it) if nxt is not None else None

        def mm(a, b):
            return jnp.dot(a, b, preferred_element_type=jnp.float32)

        cnt = jnp.clip((cA_r[...] + cB_r[...])[:, 0:1], 1.0, None)
        mean = (pA_r[...] + pB_r[...]) / cnt
        out = mm(mean, Wl_r[...]) + bl_r[...][None, :] + mm(x_r[...], Wr_r[...]) + br_r[...][None, :]
        nrm = jnp.sqrt(jnp.sum(out * out, axis=1, keepdims=True))
        h = out / jnp.clip(nrm, 1e-12, None)
        if ln is not None:
            h = h * jax.nn.sigmoid(h)
            mu = jnp.mean(h, axis=1, keepdims=True)
            var = jnp.mean((h - mu) ** 2, axis=1, keepdims=True)
            h = (h - mu) / jnp.sqrt(var + 1e-5) * g_r[...][None, :] + b_r[...][None, :]
        h_out[...] = h
        if nxt is not None:
            xp_out[...] = jax.nn.relu(mm(h, wp_r[...]) + bp_r[...][None, :])

    res = pl.pallas_call(body, out_shape=outs)(*ins)
    return res if nxt is not None else (res[0], None)


_COS_TH = np.concatenate(
    [np.cos(np.arange(1, 128, dtype=np.float64) * np.pi / 256.0), [-2.0]]
).astype(np.float32).reshape(1, 128)


def kernel(vertices, segments, segment_edges, segment_mask, segment_edges_mask, params):
    B, NV, _ = vertices.shape
    NL = segments.shape[1]
    E = segment_edges.shape[1]
    N = B * NL
    NE = 2 * B * E

    seg = jnp.where(segment_mask[..., None], segments, 0)
    se0 = jnp.where(segment_edges_mask[..., None], segment_edges, 0)

    vpad = jnp.pad(vertices.reshape(B * NV, 3), ((0, 0), (0, 13)))
    boffV = (jnp.arange(B) * NV)[:, None]
    iA = (seg[..., 0] + boffV).reshape(-1).astype(jnp.int32)
    iB = (seg[..., 1] + boffV).reshape(-1).astype(jnp.int32)
    g1 = _sc_gather(vpad, jnp.concatenate([iA, iB]))
    vAB = jnp.concatenate([g1[:N, :3], g1[N:, :3]], axis=1)
    segrows = jnp.pad(vAB, ((0, 0), (0, 10)))

    boffL = (jnp.arange(B) * NL)[:, None]
    e0 = (se0[..., 0] + boffL).reshape(-1).astype(jnp.int32)
    e1 = (se0[..., 1] + boffL).reshape(-1).astype(jnp.int32)
    g2 = _sc_gather(segrows, jnp.concatenate([e0, e1]))
    ep0, ep1 = g2[:B * E], g2[B * E:]

    av = _tc_angle(ep0, ep1, params['angle_tab'], jnp.asarray(_COS_TH))
    nbins_a = 4224
    boffA = (jnp.arange(B) * (NL + 1))[:, None]
    b0 = (se0[..., 0] + boffA).reshape(-1).astype(jnp.int32)
    b1 = (se0[..., 1] + boffA).reshape(-1).astype(jnp.int32)
    ea = jnp.arange(B * E, dtype=jnp.int32)
    pa = _sc_segsum(av, jnp.concatenate([ea, ea]), jnp.concatenate([b0, b1]), nbins_a)
    ar = pa[:, :B * (NL + 1)].reshape(_NC, B, NL + 1, 32)[:, :, :NL].reshape(_NC, N, 32)

    p = params
    x, xp = _tc_embed(segrows, ar[0], ar[1], p['coor_tab'], p['length_tab'],
                      p['midp_tab'], p['dir_tab'], p['Win'], p['bin'],
                      p['sage'][0]['Wp'], p['sage'][0]['bp'])

    src = jnp.concatenate([e0, e1])
    dst = jnp.concatenate([e1, e0])
    ones16 = jnp.ones((8, 16), jnp.float32)
    cp = _sc_segsum(ones16, jnp.zeros((NE,), jnp.int32), dst, N)
    h = x
    nlayers = len(p['sage'])
    for li in range(nlayers):
        pl_parts = _sc_segsum(xp, src, dst, N)
        sp = p['sage'][li]
        ln = (p['ln_g'], p['ln_b']) if li == 0 else None
        nxt = None
        if li + 1 < nlayers:
            nxt = (p['sage'][li + 1]['Wp'], p['sage'][li + 1]['bp'])
        h, xp = _tc_sage(h, pl_parts[0], pl_parts[1], cp[0], cp[1],
                         sp['Wl'], sp['bl'], sp['Wr'], sp['br'], ln=ln, nxt=nxt)
    return h.reshape(B, NL, 384)


# speedup vs baseline: 4.6236x; 1.0983x over previous
import numpy as np
import jax
import jax.numpy as jnp
from jax import lax
from jax.experimental import pallas as pl
from jax.experimental.pallas import tpu as pltpu
from jax.experimental.pallas import tpu_sc as plsc

_NC, _NS, _CH = 2, 16, 128
_NW = _NC * _NS


def _sc_mesh():
    return plsc.VectorSubcoreMesh(core_axis_name="c", subcore_axis_name="s")


def _sc_gather(table, idx):
    V, W = table.shape
    N = idx.shape[0]
    nch = N // (_NW * _CH)
    idx4 = idx.reshape(_NC, _NS, nch, _CH)

    def body(table_hbm, idx_hbm, out_hbm, idx_v, rows_v, sem0, sem1):
        c = lax.axis_index("c")
        s = lax.axis_index("s")
        sems = (sem0, sem1)
        pltpu.sync_copy(idx_hbm.at[c, s], idx_v)
        cps = [None, None]
        cps[0] = pltpu.async_copy(table_hbm.at[idx_v.at[0]], rows_v.at[0], sems[0])
        for k in range(nch):
            slot = k % 2
            if k + 1 < nch:
                nslot = (k + 1) % 2
                cps[nslot] = pltpu.async_copy(
                    table_hbm.at[idx_v.at[k + 1]], rows_v.at[nslot], sems[nslot])
            cps[slot].wait()
            base = ((c * _NS + s) * nch + k) * _CH
            pltpu.sync_copy(rows_v.at[slot], out_hbm.at[pl.ds(base, _CH)])

    fn = pl.kernel(
        body,
        out_type=jax.ShapeDtypeStruct((N, W), jnp.float32),
        mesh=_sc_mesh(),
        compiler_params=pltpu.CompilerParams(use_tc_tiling_on_sc=False),
        scratch_types=[
            pltpu.VMEM((nch, _CH), jnp.int32),
            pltpu.VMEM((2, _CH, W), jnp.float32),
            pltpu.SemaphoreType.DMA,
            pltpu.SemaphoreType.DMA,
        ],
    )
    return fn(table, idx4)


def _sc_segsum(vals, gidx, sidx, nbins):
    V, W = vals.shape
    dt = vals.dtype
    N = gidx.shape[0]
    wbytes = W * dt.itemsize
    ch = 64 if wbytes >= 1024 else _CH
    nch = N // (_NW * ch)
    g4 = gidx.reshape(_NC, _NS, nch, ch)
    s4 = sidx.reshape(_NC, _NS, nch, ch)
    rows_per = nbins // _NS

    def body(vals_hbm, g_hbm, s_hbm, zero_hbm, out_hbm, gi_v, si_v, rows_v, acc, sem0, sem1):
        c = lax.axis_index("c")
        s = lax.axis_index("s")
        sems = (sem0, sem1)
        r0 = s * rows_per
        pltpu.sync_copy(g_hbm.at[c, s], gi_v)
        pltpu.sync_copy(s_hbm.at[c, s], si_v)
        cps = [None, None]
        cps[0] = pltpu.async_copy(vals_hbm.at[gi_v.at[0]], rows_v.at[0], sems[0])
        pltpu.sync_copy(zero_hbm.at[pl.ds(r0, rows_per)], acc.at[pl.ds(r0, rows_per)])
        plsc.subcore_barrier()
        for k in range(nch):
            slot = k % 2
            if k + 1 < nch:
                nslot = (k + 1) % 2
                cps[nslot] = pltpu.async_copy(
                    vals_hbm.at[gi_v.at[k + 1]], rows_v.at[nslot], sems[nslot])
            cps[slot].wait()
            pltpu.sync_copy(rows_v.at[slot], acc.at[si_v.at[k]], add=True)
        plsc.subcore_barrier()
        pltpu.sync_copy(acc.at[pl.ds(r0, rows_per)], out_hbm.at[c, pl.ds(r0, rows_per)])

    zero = jnp.zeros((nbins, W), dt)
    fn = pl.kernel(
        body,
        out_type=jax.ShapeDtypeStruct((_NC, nbins, W), dt),
        mesh=_sc_mesh(),
        compiler_params=pltpu.CompilerParams(use_tc_tiling_on_sc=False),
        scratch_types=[
            pltpu.VMEM((nch, ch), jnp.int32),
            pltpu.VMEM((nch, ch), jnp.int32),
            pltpu.VMEM((2, ch, W), dt),
            pltpu.VMEM_SHARED((nbins, W), dt),
            pltpu.SemaphoreType.DMA,
            pltpu.SemaphoreType.DMA,
        ],
    )
    return fn(vals, g4, s4, zero)


def _tc_angle(ep0, ep1, angle_tab, cos_th):
    Ne = ep0.shape[0]

    def body(e0_ref, e1_ref, tab_ref, th_ref, out_ref):
        def dirn(e):
            d = e[:, 3:6] - e[:, 0:3]
            n = jnp.sqrt(jnp.sum(d * d, axis=1, keepdims=True) + 1e-12)
            return d / jnp.clip(n, 1e-8, None)

        c = jnp.abs(jnp.sum(dirn(e0_ref[...]) * dirn(e1_ref[...]), axis=1, keepdims=True))
        c = jnp.clip(c, 0.0, 1.0 - 1e-6)
        idx = jnp.sum((c <= th_ref[...]).astype(jnp.int32), axis=1)
        oh = (idx[:, None] == lax.broadcasted_iota(jnp.int32, (Ne, 128), 1)).astype(jnp.float32)
        vals = jnp.dot(oh, tab_ref[...], preferred_element_type=jnp.float32)
        out_ref[...] = jnp.concatenate([vals, jnp.ones((Ne, 16), jnp.float32)], axis=1)

    return pl.pallas_call(
        body, out_shape=jax.ShapeDtypeStruct((Ne, 32), jnp.float32)
    )(ep0, ep1, angle_tab, cos_th)


def _tc_embed(segrows, angA, angB, ct, lt, mt, dt, win, bin_, wp, bp):
    N = segrows.shape[0]

    def body(sr_ref, aA_ref, aB_ref, ct_ref, lt_ref, mt_ref, dt_ref,
             win_ref, bin_ref, wp_ref, bp_ref, x_out, xp_out):
        sr = sr_ref[...]
        v0 = sr[:, 0:3]
        v1 = sr[:, 3:6]
        midps = 0.5 * (v0 + v1)
        vec = v1 - v0
        length = jnp.sqrt(jnp.sum(vec * vec, axis=1, keepdims=True) + 1e-12)
        dirs = vec / jnp.clip(length, 1e-8, None)

        def disc(t, lo, hi):
            t = (t - lo) / (hi - lo) * 128.0 - 0.5
            return jnp.clip(jnp.round(t), 0, 127).astype(jnp.int32)

        def oh(ix):
            return (ix[:, None] == lax.broadcasted_iota(jnp.int32, (N, 128), 1)
                    ).astype(jnp.float32)

        def mm(a, b):
            return jnp.dot(a, b, preferred_element_type=jnp.float32)

        win = win_ref[...]
        x = jnp.broadcast_to(bin_ref[...][None, :], (N, 192))
        ixc = disc(jnp.concatenate([v0, v1], axis=1), -1.0, 1.0)
        ct_t = ct_ref[...]
        for k in range(6):
            x = x + mm(oh(ixc[:, k]), mm(ct_t, win[64 * k:64 * (k + 1), :]))
        ixl = disc(length, 0.0, 2.0)[:, 0]
        x = x + mm(oh(ixl), mm(lt_ref[...], win[384:400, :]))
        ixd = disc(dirs, -1.0, 1.0)
        dt_t = dt_ref[...]
        for k in range(3):
            x = x + mm(oh(ixd[:, k]), mm(dt_t, win[400 + 64 * k:400 + 64 * (k + 1), :]))
        aa = aA_ref[...] + aB_ref[...]
        amean = aa[:, 0:16] / jnp.clip(aa[:, 16:17], 1.0, None)
        x = x + mm(amean, win[592:608, :])
        ixm = disc(midps, -1.0, 1.0)
        mt_t = mt_ref[...]
        for k in range(3):
            x = x + mm(oh(ixm[:, k]), mm(mt_t, win[608 + 16 * k:608 + 16 * (k + 1), :]))
        x_out[...] = x
        xp_out[...] = jax.nn.relu(mm(x, wp_ref[...]) + bp_ref[...][None, :])

    return pl.pallas_call(
        body,
        out_shape=[
            jax.ShapeDtypeStruct((N, 192), jnp.float32),
            jax.ShapeDtypeStruct((N, wp.shape[1]), jnp.float32),
        ],
    )(segrows, angA, angB, ct, lt, mt, dt, win, bin_, wp, bp)


def _tc_sage(x, pA, pB, cA, cB, Wl, bl, Wr, br, ln=None, nxt=None):
    N, _ = x.shape
    o = Wl.shape[1]
    outs = [jax.ShapeDtypeStruct((N, o), jnp.float32)]
    ins = [x, pA, pB, cA, cB, Wl, bl, Wr, br]
    if ln is not None:
        ins += [ln[0], ln[1]]
    if nxt is not None:
        ins += [nxt[0], nxt[1]]
        outs.append(jax.ShapeDtypeStruct((N, nxt[0].shape[1]), jnp.float32))

    def body(*refs):
        it = iter(refs)
        x_r, pA_r, pB_r, cA_r, cB_r, Wl_r, bl_r, Wr_r, br_r = (next(it) for _ in range(9))
        if ln is not None:
            g_r, b_r = next(it), next(it)
        if nxt is not None:
            wp_r, bp_r = next(it), next(it)
        h_out = next(it)
        xp_out = next(it) if nxt is not None else None

        def mm(a, b):
            return jnp.dot(a, b, preferred_element_type=jnp.float32)

        cnt = jnp.clip((cA_r[...] + cB_r[...])[:, 0:1], 1.0, None)
        mean = (pA_r[...].astype(jnp.float32) + pB_r[...].astype(jnp.float32)) / cnt
        out = mm(mean, Wl_r[...]) + bl_r[...][None, :] + mm(x_r[...], Wr_r[...]) + br_r[...][None, :]
        nrm = jnp.sqrt(jnp.sum(out * out, axis=1, keepdims=True))
        h = out / jnp.clip(nrm, 1e-12, None)
        if ln is not None:
            h = h * jax.nn.sigmoid(h)
            mu = jnp.mean(h, axis=1, keepdims=True)
            var = jnp.mean((h - mu) ** 2, axis=1, keepdims=True)
            h = (h - mu) / jnp.sqrt(var + 1e-5) * g_r[...][None, :] + b_r[...][None, :]
        h_out[...] = h
        if nxt is not None:
            xp_out[...] = jax.nn.relu(mm(h, wp_r[...]) + bp_r[...][None, :])

    res = pl.pallas_call(body, out_shape=outs)(*ins)
    return res if nxt is not None else (res[0], None)


_COS_TH = np.concatenate(
    [np.cos(np.arange(1, 128, dtype=np.float64) * np.pi / 256.0), [-2.0]]
).astype(np.float32).reshape(1, 128)


def kernel(vertices, segments, segment_edges, segment_mask, segment_edges_mask, params):
    B, NV, _ = vertices.shape
    NL = segments.shape[1]
    E = segment_edges.shape[1]
    N = B * NL
    NE = 2 * B * E

    seg = jnp.where(segment_mask[..., None], segments, 0)
    se0 = jnp.where(segment_edges_mask[..., None], segment_edges, 0)

    vpad = jnp.pad(vertices.reshape(B * NV, 3), ((0, 0), (0, 13)))
    boffV = (jnp.arange(B) * NV)[:, None]
    iA = (seg[..., 0] + boffV).reshape(-1).astype(jnp.int32)
    iB = (seg[..., 1] + boffV).reshape(-1).astype(jnp.int32)
    g1 = _sc_gather(vpad, jnp.concatenate([iA, iB]))
    vAB = jnp.concatenate([g1[:N, :3], g1[N:, :3]], axis=1)
    segrows = jnp.pad(vAB, ((0, 0), (0, 10)))

    boffL = (jnp.arange(B) * NL)[:, None]
    e0 = (se0[..., 0] + boffL).reshape(-1).astype(jnp.int32)
    e1 = (se0[..., 1] + boffL).reshape(-1).astype(jnp.int32)
    g2 = _sc_gather(segrows, jnp.concatenate([e0, e1]))
    ep0, ep1 = g2[:B * E], g2[B * E:]

    av = _tc_angle(ep0, ep1, params['angle_tab'], jnp.asarray(_COS_TH))
    nbins_a = 4224
    boffA = (jnp.arange(B) * (NL + 1))[:, None]
    b0 = (se0[..., 0] + boffA).reshape(-1).astype(jnp.int32)
    b1 = (se0[..., 1] + boffA).reshape(-1).astype(jnp.int32)
    ea = jnp.arange(B * E, dtype=jnp.int32)
    pa = _sc_segsum(av, jnp.concatenate([ea, ea]), jnp.concatenate([b0, b1]), nbins_a)
    ar = pa[:, :B * (NL + 1)].reshape(_NC, B, NL + 1, 32)[:, :, :NL].reshape(_NC, N, 32)

    p = params
    x, xp = _tc_embed(segrows, ar[0], ar[1], p['coor_tab'], p['length_tab'],
                      p['midp_tab'], p['dir_tab'], p['Win'], p['bin'],
                      p['sage'][0]['Wp'], p['sage'][0]['bp'])

    src = jnp.concatenate([e0, e1])
    dst = jnp.concatenate([e1, e0])
    ones16 = jnp.ones((8, 16), jnp.float32)
    cp = _sc_segsum(ones16, jnp.zeros((NE,), jnp.int32), dst, N)
    h = x
    nlayers = len(p['sage'])
    for li in range(nlayers):
        pl_parts = _sc_segsum(xp.astype(jnp.bfloat16), src, dst, N)
        sp = p['sage'][li]
        ln = (p['ln_g'], p['ln_b']) if li == 0 else None
        nxt = None
        if li + 1 < nlayers:
            nxt = (p['sage'][li + 1]['Wp'], p['sage'][li + 1]['bp'])
        h, xp = _tc_sage(h, pl_parts[0], pl_parts[1], cp[0], cp[1],
                         sp['Wl'], sp['bl'], sp['Wr'], sp['br'], ln=ln, nxt=nxt)
    return h.reshape(B, NL, 384)


# speedup vs baseline: 5.4457x; 1.1778x over previous
import numpy as np
import jax
import jax.numpy as jnp
from jax import lax
from jax.experimental import pallas as pl
from jax.experimental.pallas import tpu as pltpu
from jax.experimental.pallas import tpu_sc as plsc

_NC, _NS, _CH = 2, 16, 128
_NW = _NC * _NS


def _sc_mesh():
    return plsc.VectorSubcoreMesh(core_axis_name="c", subcore_axis_name="s")


def _sc_gather(table, idx):
    V, W = table.shape
    N = idx.shape[0]
    nch = N // (_NW * _CH)
    idx4 = idx.reshape(_NC, _NS, nch, _CH)

    def body(table_hbm, idx_hbm, out_hbm, idx_v, rows_v, sem0, sem1):
        c = lax.axis_index("c")
        s = lax.axis_index("s")
        sems = (sem0, sem1)
        pltpu.sync_copy(idx_hbm.at[c, s], idx_v)
        cps = [None, None]
        cps[0] = pltpu.async_copy(table_hbm.at[idx_v.at[0]], rows_v.at[0], sems[0])
        for k in range(nch):
            slot = k % 2
            if k + 1 < nch:
                nslot = (k + 1) % 2
                cps[nslot] = pltpu.async_copy(
                    table_hbm.at[idx_v.at[k + 1]], rows_v.at[nslot], sems[nslot])
            cps[slot].wait()
            base = ((c * _NS + s) * nch + k) * _CH
            pltpu.sync_copy(rows_v.at[slot], out_hbm.at[pl.ds(base, _CH)])

    fn = pl.kernel(
        body,
        out_type=jax.ShapeDtypeStruct((N, W), jnp.float32),
        mesh=_sc_mesh(),
        compiler_params=pltpu.CompilerParams(use_tc_tiling_on_sc=False),
        scratch_types=[
            pltpu.VMEM((nch, _CH), jnp.int32),
            pltpu.VMEM((2, _CH, W), jnp.float32),
            pltpu.SemaphoreType.DMA,
            pltpu.SemaphoreType.DMA,
        ],
    )
    return fn(table, idx4)


def _sc_segsum(vals, gidx, sidx, nbins):
    V, W = vals.shape
    dt = vals.dtype
    N = sidx.shape[0]
    wbytes = W * dt.itemsize
    ch = 64 if wbytes >= 1024 else _CH
    nch = N // (_NW * ch)
    s4 = sidx.reshape(_NC, _NS, nch, ch)
    rows_per = nbins // _NS
    zero = jnp.zeros((nbins, W), dt)

    if gidx is None:
        def body(vals_hbm, s_hbm, zero_hbm, out_hbm, si_v, rows_v, acc):
            c = lax.axis_index("c")
            s = lax.axis_index("s")
            r0 = s * rows_per
            pltpu.sync_copy(s_hbm.at[c, s], si_v)
            pltpu.sync_copy(vals_hbm, rows_v)
            pltpu.sync_copy(zero_hbm.at[pl.ds(r0, rows_per)], acc.at[pl.ds(r0, rows_per)])
            plsc.subcore_barrier()
            for k in range(nch):
                pltpu.sync_copy(rows_v, acc.at[si_v.at[k]], add=True)
            plsc.subcore_barrier()
            pltpu.sync_copy(acc.at[pl.ds(r0, rows_per)], out_hbm.at[c, pl.ds(r0, rows_per)])

        fn = pl.kernel(
            body,
            out_type=jax.ShapeDtypeStruct((_NC, nbins, W), dt),
            mesh=_sc_mesh(),
            compiler_params=pltpu.CompilerParams(use_tc_tiling_on_sc=False),
            scratch_types=[
                pltpu.VMEM((nch, ch), jnp.int32),
                pltpu.VMEM((ch, W), dt),
                pltpu.VMEM_SHARED((nbins, W), dt),
            ],
        )
        return fn(vals, s4, zero)

    g4 = gidx.reshape(_NC, _NS, nch, ch)

    def body(vals_hbm, g_hbm, s_hbm, zero_hbm, out_hbm, gi_v, si_v, rows_v, acc, sem0, sem1):
        c = lax.axis_index("c")
        s = lax.axis_index("s")
        sems = (sem0, sem1)
        r0 = s * rows_per
        pltpu.sync_copy(g_hbm.at[c, s], gi_v)
        pltpu.sync_copy(s_hbm.at[c, s], si_v)
        cps = [None, None]
        cps[0] = pltpu.async_copy(vals_hbm.at[gi_v.at[0]], rows_v.at[0], sems[0])
        pltpu.sync_copy(zero_hbm.at[pl.ds(r0, rows_per)], acc.at[pl.ds(r0, rows_per)])
        plsc.subcore_barrier()
        for k in range(nch):
            slot = k % 2
            if k + 1 < nch:
                nslot = (k + 1) % 2
                cps[nslot] = pltpu.async_copy(
                    vals_hbm.at[gi_v.at[k + 1]], rows_v.at[nslot], sems[nslot])
            cps[slot].wait()
            pltpu.sync_copy(rows_v.at[slot], acc.at[si_v.at[k]], add=True)
        plsc.subcore_barrier()
        pltpu.sync_copy(acc.at[pl.ds(r0, rows_per)], out_hbm.at[c, pl.ds(r0, rows_per)])

    fn = pl.kernel(
        body,
        out_type=jax.ShapeDtypeStruct((_NC, nbins, W), dt),
        mesh=_sc_mesh(),
        compiler_params=pltpu.CompilerParams(use_tc_tiling_on_sc=False),
        scratch_types=[
            pltpu.VMEM((nch, ch), jnp.int32),
            pltpu.VMEM((nch, ch), jnp.int32),
            pltpu.VMEM((2, ch, W), dt),
            pltpu.VMEM_SHARED((nbins, W), dt),
            pltpu.SemaphoreType.DMA,
            pltpu.SemaphoreType.DMA,
        ],
    )
    return fn(vals, g4, s4, zero)


def _tc_angle(ep0, ep1, angle_tab, cos_th):
    Ne = ep0.shape[0]

    def body(e0_ref, e1_ref, tab_ref, th_ref, out_ref):
        def dirn(e):
            d = e[:, 3:6] - e[:, 0:3]
            n = jnp.sqrt(jnp.sum(d * d, axis=1, keepdims=True) + 1e-12)
            return d / jnp.clip(n, 1e-8, None)

        c = jnp.abs(jnp.sum(dirn(e0_ref[...]) * dirn(e1_ref[...]), axis=1, keepdims=True))
        c = jnp.clip(c, 0.0, 1.0 - 1e-6)
        idx = jnp.sum((c <= th_ref[...]).astype(jnp.int32), axis=1)
        oh = (idx[:, None] == lax.broadcasted_iota(jnp.int32, (Ne, 128), 1)).astype(jnp.float32)
        vals = jnp.dot(oh, tab_ref[...], preferred_element_type=jnp.float32)
        out_ref[...] = jnp.concatenate([vals, jnp.ones((Ne, 16), jnp.float32)], axis=1)

    return pl.pallas_call(
        body, out_shape=jax.ShapeDtypeStruct((Ne, 32), jnp.float32)
    )(ep0, ep1, angle_tab, cos_th)


def _tc_embed(segrows, angA, angB, ct, lt, mt, dt, win, bin_, wp, bp):
    N = segrows.shape[0]

    def body(sr_ref, aA_ref, aB_ref, ct_ref, lt_ref, mt_ref, dt_ref,
             win_ref, bin_ref, wp_ref, bp_ref, x_out, xp_out):
        sr = sr_ref[...]
        v0 = sr[:, 0:3]
        v1 = sr[:, 3:6]
        midps = 0.5 * (v0 + v1)
        vec = v1 - v0
        length = jnp.sqrt(jnp.sum(vec * vec, axis=1, keepdims=True) + 1e-12)
        dirs = vec / jnp.clip(length, 1e-8, None)

        def disc(t, lo, hi):
            t = (t - lo) / (hi - lo) * 128.0 - 0.5
            return jnp.clip(jnp.round(t), 0, 127).astype(jnp.int32)

        def oh(ix):
            return (ix[:, None] == lax.broadcasted_iota(jnp.int32, (N, 128), 1)
                    ).astype(jnp.float32)

        def mm(a, b):
            return jnp.dot(a, b, preferred_element_type=jnp.float32)

        win = win_ref[...]
        x = jnp.broadcast_to(bin_ref[...][None, :], (N, 192))
        ixc = disc(jnp.concatenate([v0, v1], axis=1), -1.0, 1.0)
        ct_t = ct_ref[...]
        for k in range(6):
            x = x + mm(oh(ixc[:, k]), mm(ct_t, win[64 * k:64 * (k + 1), :]))
        ixl = disc(length, 0.0, 2.0)[:, 0]
        x = x + mm(oh(ixl), mm(lt_ref[...], win[384:400, :]))
        ixd = disc(dirs, -1.0, 1.0)
        dt_t = dt_ref[...]
        for k in range(3):
            x = x + mm(oh(ixd[:, k]), mm(dt_t, win[400 + 64 * k:400 + 64 * (k + 1), :]))
        aa = aA_ref[...].astype(jnp.float32) + aB_ref[...].astype(jnp.float32)
        amean = aa[:, 0:16] / jnp.clip(aa[:, 16:17], 1.0, None)
        x = x + mm(amean, win[592:608, :])
        ixm = disc(midps, -1.0, 1.0)
        mt_t = mt_ref[...]
        for k in range(3):
            x = x + mm(oh(ixm[:, k]), mm(mt_t, win[608 + 16 * k:608 + 16 * (k + 1), :]))
        x_out[...] = x
        xp_out[...] = jax.nn.relu(mm(x, wp_ref[...]) + bp_ref[...][None, :])

    return pl.pallas_call(
        body,
        out_shape=[
            jax.ShapeDtypeStruct((N, 192), jnp.float32),
            jax.ShapeDtypeStruct((N, wp.shape[1]), jnp.float32),
        ],
    )(segrows, angA, angB, ct, lt, mt, dt, win, bin_, wp, bp)


def _tc_sage(x, pA, pB, cA, cB, Wl, bl, Wr, br, ln=None, nxt=None):
    N, _ = x.shape
    o = Wl.shape[1]
    outs = [jax.ShapeDtypeStruct((N, o), jnp.float32)]
    ins = [x, pA, pB, cA, cB, Wl, bl, Wr, br]
    if ln is not None:
        ins += [ln[0], ln[1]]
    if nxt is not None:
        ins += [nxt[0], nxt[1]]
        outs.append(jax.ShapeDtypeStruct((N, nxt[0].shape[1]), jnp.float32))

    def body(*refs):
        it = iter(refs)
        x_r, pA_r, pB_r, cA_r, cB_r, Wl_r, bl_r, Wr_r, br_r = (next(it) for _ in range(9))
        if ln is not None:
            g_r, b_r = next(it), next(it)
        if nxt is not None:
            wp_r, bp_r = next(it), next(it)
        h_out = next(it)
        xp_out = next(it) if nxt is not None else None

        def mm(a, b):
            return jnp.dot(a, b, preferred_element_type=jnp.float32)

        cnt = jnp.clip((cA_r[...].astype(jnp.float32) + cB_r[...].astype(jnp.float32))[:, 0:1], 1.0, None)
        mean = (pA_r[...].astype(jnp.float32) + pB_r[...].astype(jnp.float32)) / cnt
        out = mm(mean, Wl_r[...]) + bl_r[...][None, :] + mm(x_r[...], Wr_r[...]) + br_r[...][None, :]
        nrm = jnp.sqrt(jnp.sum(out * out, axis=1, keepdims=True))
        h = out / jnp.clip(nrm, 1e-12, None)
        if ln is not None:
            h = h * jax.nn.sigmoid(h)
            mu = jnp.mean(h, axis=1, keepdims=True)
            var = jnp.mean((h - mu) ** 2, axis=1, keepdims=True)
            h = (h - mu) / jnp.sqrt(var + 1e-5) * g_r[...][None, :] + b_r[...][None, :]
        h_out[...] = h
        if nxt is not None:
            xp_out[...] = jax.nn.relu(mm(h, wp_r[...]) + bp_r[...][None, :])

    res = pl.pallas_call(body, out_shape=outs)(*ins)
    return res if nxt is not None else (res[0], None)


_COS_TH = np.concatenate(
    [np.cos(np.arange(1, 128, dtype=np.float64) * np.pi / 256.0), [-2.0]]
).astype(np.float32).reshape(1, 128)


def kernel(vertices, segments, segment_edges, segment_mask, segment_edges_mask, params):
    B, NV, _ = vertices.shape
    NL = segments.shape[1]
    E = segment_edges.shape[1]
    N = B * NL
    NE = 2 * B * E

    seg = jnp.where(segment_mask[..., None], segments, 0)
    se0 = jnp.where(segment_edges_mask[..., None], segment_edges, 0)

    vpad = jnp.pad(vertices.reshape(B * NV, 3), ((0, 0), (0, 13)))
    boffV = (jnp.arange(B) * NV)[:, None]
    iA = (seg[..., 0] + boffV).reshape(-1).astype(jnp.int32)
    iB = (seg[..., 1] + boffV).reshape(-1).astype(jnp.int32)
    g1 = _sc_gather(vpad, jnp.concatenate([iA, iB]))
    vAB = jnp.concatenate([g1[:N, :3], g1[N:, :3]], axis=1)
    segrows = jnp.pad(vAB, ((0, 0), (0, 10)))

    boffL = (jnp.arange(B) * NL)[:, None]
    e0 = (se0[..., 0] + boffL).reshape(-1).astype(jnp.int32)
    e1 = (se0[..., 1] + boffL).reshape(-1).astype(jnp.int32)
    g2 = _sc_gather(segrows, jnp.concatenate([e0, e1]))
    ep0, ep1 = g2[:B * E], g2[B * E:]

    av = _tc_angle(ep0, ep1, params['angle_tab'], jnp.asarray(_COS_TH))
    nbins_a = 4224
    boffA = (jnp.arange(B) * (NL + 1))[:, None]
    b0 = (se0[..., 0] + boffA).reshape(-1).astype(jnp.int32)
    b1 = (se0[..., 1] + boffA).reshape(-1).astype(jnp.int32)
    ea = jnp.arange(B * E, dtype=jnp.int32)
    pa = _sc_segsum(av.astype(jnp.bfloat16), jnp.concatenate([ea, ea]),
                    jnp.concatenate([b0, b1]), nbins_a)
    ar = pa[:, :B * (NL + 1)].reshape(_NC, B, NL + 1, 32)[:, :, :NL].reshape(_NC, N, 32)

    p = params
    x, xp = _tc_embed(segrows, ar[0], ar[1], p['coor_tab'], p['length_tab'],
                      p['midp_tab'], p['dir_tab'], p['Win'], p['bin'],
                      p['sage'][0]['Wp'], p['sage'][0]['bp'])

    src = jnp.concatenate([e0, e1])
    dst = jnp.concatenate([e1, e0])
    cp = _sc_segsum(jnp.ones((_CH, 16), jnp.bfloat16), None, dst, N)
    h = x
    nlayers = len(p['sage'])
    for li in range(nlayers):
        pl_parts = _sc_segsum(xp.astype(jnp.bfloat16), src, dst, N)
        sp = p['sage'][li]
        ln = (p['ln_g'], p['ln_b']) if li == 0 else None
        nxt = None
        if li + 1 < nlayers:
            nxt = (p['sage'][li + 1]['Wp'], p['sage'][li + 1]['bp'])
        h, xp = _tc_sage(h, pl_parts[0], pl_parts[1], cp[0], cp[1],
                         sp['Wl'], sp['bl'], sp['Wr'], sp['br'], ln=ln, nxt=nxt)
    return h.reshape(B, NL, 384)


# speedup vs baseline: 5.4540x; 1.0015x over previous
import numpy as np
import jax
import jax.numpy as jnp
from jax import lax
from jax.experimental import pallas as pl
from jax.experimental.pallas import tpu as pltpu
from jax.experimental.pallas import tpu_sc as plsc

_NC, _NS, _CH = 2, 16, 128
_NW = _NC * _NS


def _sc_mesh():
    return plsc.VectorSubcoreMesh(core_axis_name="c", subcore_axis_name="s")


def _sc_gather(table, idx):
    V, W = table.shape
    N = idx.shape[0]
    nch = N // (_NW * _CH)
    idx4 = idx.reshape(_NC, _NS, nch, _CH)

    def body(table_hbm, idx_hbm, out_hbm, idx_v, rows_v, sem0, sem1):
        c = lax.axis_index("c")
        s = lax.axis_index("s")
        sems = (sem0, sem1)
        pltpu.sync_copy(idx_hbm.at[c, s], idx_v)
        cps = [None, None]
        cps[0] = pltpu.async_copy(table_hbm.at[idx_v.at[0]], rows_v.at[0], sems[0])
        for k in range(nch):
            slot = k % 2
            if k + 1 < nch:
                nslot = (k + 1) % 2
                cps[nslot] = pltpu.async_copy(
                    table_hbm.at[idx_v.at[k + 1]], rows_v.at[nslot], sems[nslot])
            cps[slot].wait()
            base = ((c * _NS + s) * nch + k) * _CH
            pltpu.sync_copy(rows_v.at[slot], out_hbm.at[pl.ds(base, _CH)])

    fn = pl.kernel(
        body,
        out_type=jax.ShapeDtypeStruct((N, W), jnp.float32),
        mesh=_sc_mesh(),
        compiler_params=pltpu.CompilerParams(use_tc_tiling_on_sc=False),
        scratch_types=[
            pltpu.VMEM((nch, _CH), jnp.int32),
            pltpu.VMEM((2, _CH, W), jnp.float32),
            pltpu.SemaphoreType.DMA,
            pltpu.SemaphoreType.DMA,
        ],
    )
    return fn(table, idx4)


def _sc_segsum(vals, gidx, sidx, nbins):
    V, W = vals.shape
    dt = vals.dtype
    N = sidx.shape[0]
    wbytes = W * dt.itemsize
    ch = 64 if wbytes >= 1024 else _CH
    nch = N // (_NW * ch)
    s4 = sidx.reshape(_NC, _NS, nch, ch)
    rows_per = nbins // _NS
    zero = jnp.zeros((nbins, W), dt)

    if gidx is None:
        def body(vals_hbm, s_hbm, zero_hbm, out_hbm, si_v, rows_v, acc):
            c = lax.axis_index("c")
            s = lax.axis_index("s")
            r0 = s * rows_per
            pltpu.sync_copy(s_hbm.at[c, s], si_v)
            pltpu.sync_copy(vals_hbm, rows_v)
            pltpu.sync_copy(zero_hbm.at[pl.ds(r0, rows_per)], acc.at[pl.ds(r0, rows_per)])
            plsc.subcore_barrier()
            for k in range(nch):
                pltpu.sync_copy(rows_v, acc.at[si_v.at[k]], add=True)
            plsc.subcore_barrier()
            pltpu.sync_copy(acc.at[pl.ds(r0, rows_per)], out_hbm.at[c, pl.ds(r0, rows_per)])

        fn = pl.kernel(
            body,
            out_type=jax.ShapeDtypeStruct((_NC, nbins, W), dt),
            mesh=_sc_mesh(),
            compiler_params=pltpu.CompilerParams(use_tc_tiling_on_sc=False),
            scratch_types=[
                pltpu.VMEM((nch, ch), jnp.int32),
                pltpu.VMEM((ch, W), dt),
                pltpu.VMEM_SHARED((nbins, W), dt),
            ],
        )
        return fn(vals, s4, zero)

    g4 = gidx.reshape(_NC, _NS, nch, ch)

    def body(vals_hbm, g_hbm, s_hbm, zero_hbm, out_hbm, gi_v, si_v, rows_v, acc, sem0, sem1):
        c = lax.axis_index("c")
        s = lax.axis_index("s")
        sems = (sem0, sem1)
        r0 = s * rows_per
        pltpu.sync_copy(g_hbm.at[c, s], gi_v)
        pltpu.sync_copy(s_hbm.at[c, s], si_v)
        cps = [None, None]
        cps[0] = pltpu.async_copy(vals_hbm.at[gi_v.at[0]], rows_v.at[0], sems[0])
        pltpu.sync_copy(zero_hbm.at[pl.ds(r0, rows_per)], acc.at[pl.ds(r0, rows_per)])
        plsc.subcore_barrier()
        for k in range(nch):
            slot = k % 2
            if k + 1 < nch:
                nslot = (k + 1) % 2
                cps[nslot] = pltpu.async_copy(
                    vals_hbm.at[gi_v.at[k + 1]], rows_v.at[nslot], sems[nslot])
            cps[slot].wait()
            pltpu.sync_copy(rows_v.at[slot], acc.at[si_v.at[k]], add=True)
        plsc.subcore_barrier()
        pltpu.sync_copy(acc.at[pl.ds(r0, rows_per)], out_hbm.at[c, pl.ds(r0, rows_per)])

    fn = pl.kernel(
        body,
        out_type=jax.ShapeDtypeStruct((_NC, nbins, W), dt),
        mesh=_sc_mesh(),
        compiler_params=pltpu.CompilerParams(use_tc_tiling_on_sc=False),
        scratch_types=[
            pltpu.VMEM((nch, ch), jnp.int32),
            pltpu.VMEM((nch, ch), jnp.int32),
            pltpu.VMEM((2, ch, W), dt),
            pltpu.VMEM_SHARED((nbins, W), dt),
            pltpu.SemaphoreType.DMA,
            pltpu.SemaphoreType.DMA,
        ],
    )
    return fn(vals, g4, s4, zero)


def _tc_angle(ep0, ep1, angle_tab, cos_th):
    Ne = ep0.shape[0]

    def body(e0_ref, e1_ref, tab_ref, th_ref, out_ref):
        def dirn(e):
            d = e[:, 3:6] - e[:, 0:3]
            n = jnp.sqrt(jnp.sum(d * d, axis=1, keepdims=True) + 1e-12)
            return d / jnp.clip(n, 1e-8, None)

        c = jnp.abs(jnp.sum(dirn(e0_ref[...]) * dirn(e1_ref[...]), axis=1, keepdims=True))
        c = jnp.clip(c, 0.0, 1.0 - 1e-6)
        idx = jnp.sum((c <= th_ref[...]).astype(jnp.int32), axis=1)
        oh = (idx[:, None] == lax.broadcasted_iota(jnp.int32, (Ne, 128), 1)).astype(jnp.float32)
        vals = jnp.dot(oh, tab_ref[...], preferred_element_type=jnp.float32)
        out_ref[...] = jnp.concatenate([vals, jnp.ones((Ne, 16), jnp.float32)], axis=1)

    return pl.pallas_call(
        body, out_shape=jax.ShapeDtypeStruct((Ne, 32), jnp.float32)
    )(ep0, ep1, angle_tab, cos_th)


def _tc_embed(segrows, angA, angB, ct, lt, mt, dt, win, bin_, wp, bp):
    N = segrows.shape[0]

    def body(sr_ref, aA_ref, aB_ref, ct_ref, lt_ref, mt_ref, dt_ref,
             win_ref, bin_ref, wp_ref, bp_ref, x_out, xp_out):
        sr = sr_ref[...]
        v0 = sr[:, 0:3]
        v1 = sr[:, 3:6]
        midps = 0.5 * (v0 + v1)
        vec = v1 - v0
        length = jnp.sqrt(jnp.sum(vec * vec, axis=1, keepdims=True) + 1e-12)
        dirs = vec / jnp.clip(length, 1e-8, None)

        def disc(t, lo, hi):
            t = (t - lo) / (hi - lo) * 128.0 - 0.5
            return jnp.clip(jnp.round(t), 0, 127).astype(jnp.int32)

        def oh(ix):
            return (ix[:, None] == lax.broadcasted_iota(jnp.int32, (N, 128), 1)
                    ).astype(jnp.float32)

        def mm(a, b):
            return jnp.dot(a, b, preferred_element_type=jnp.float32)

        win = win_ref[...]
        x = jnp.broadcast_to(bin_ref[...][None, :], (N, 192))
        ixc = disc(jnp.concatenate([v0, v1], axis=1), -1.0, 1.0)
        ct_t = ct_ref[...]
        for k in range(6):
            x = x + mm(oh(ixc[:, k]), mm(ct_t, win[64 * k:64 * (k + 1), :]))
        ixl = disc(length, 0.0, 2.0)[:, 0]
        x = x + mm(oh(ixl), mm(lt_ref[...], win[384:400, :]))
        ixd = disc(dirs, -1.0, 1.0)
        dt_t = dt_ref[...]
        for k in range(3):
            x = x + mm(oh(ixd[:, k]), mm(dt_t, win[400 + 64 * k:400 + 64 * (k + 1), :]))
        aa = aA_ref[...].astype(jnp.float32) + aB_ref[...].astype(jnp.float32)
        amean = aa[:, 0:16] / jnp.clip(aa[:, 16:17], 1.0, None)
        x = x + mm(amean, win[592:608, :])
        ixm = disc(midps, -1.0, 1.0)
        mt_t = mt_ref[...]
        for k in range(3):
            x = x + mm(oh(ixm[:, k]), mm(mt_t, win[608 + 16 * k:608 + 16 * (k + 1), :]))
        x_out[...] = x
        xp_out[...] = jax.nn.relu(mm(x, wp_ref[...]) + bp_ref[...][None, :])

    return pl.pallas_call(
        body,
        out_shape=[
            jax.ShapeDtypeStruct((N, 192), jnp.float32),
            jax.ShapeDtypeStruct((N, wp.shape[1]), jnp.float32),
        ],
    )(segrows, angA, angB, ct, lt, mt, dt, win, bin_, wp, bp)


def _tc_sage(x, pA, pB, cA, cB, Wl, bl, Wr, br, ln=None, nxt=None):
    N, _ = x.shape
    o = Wl.shape[1]
    outs = [jax.ShapeDtypeStruct((N, o), jnp.float32)]
    ins = [x, pA, pB, cA, cB, Wl, bl, Wr, br]
    if ln is not None:
        ins += [ln[0], ln[1]]
    if nxt is not None:
        ins += [nxt[0], nxt[1]]
        outs.append(jax.ShapeDtypeStruct((N, nxt[0].shape[1]), jnp.float32))

    def body(*refs):
        it = iter(refs)
        x_r, pA_r, pB_r, cA_r, cB_r, Wl_r, bl_r, Wr_r, br_r = (next(it) for _ in range(9))
        if ln is not None:
            g_r, b_r = next(it), next(it)
        if nxt is not None:
            wp_r, bp_r = next(it), next(it)
        h_out = next(it)
        xp_out = next(it) if nxt is not None else None

        def mm(a, b):
            return jnp.dot(a, b, preferred_element_type=jnp.float32)

        cnt = jnp.clip((cA_r[...].astype(jnp.float32) + cB_r[...].astype(jnp.float32))[:, 0:1], 1.0, None)
        mean = (pA_r[...].astype(jnp.float32) + pB_r[...].astype(jnp.float32)) / cnt
        out = mm(mean, Wl_r[...]) + bl_r[...][None, :] + mm(x_r[...], Wr_r[...]) + br_r[...][None, :]
        nrm = jnp.sqrt(jnp.sum(out * out, axis=1, keepdims=True))
        h = out / jnp.clip(nrm, 1e-12, None)
        if ln is not None:
            h = h * jax.nn.sigmoid(h)
            mu = jnp.mean(h, axis=1, keepdims=True)
            var = jnp.mean((h - mu) ** 2, axis=1, keepdims=True)
            h = (h - mu) / jnp.sqrt(var + 1e-5) * g_r[...][None, :] + b_r[...][None, :]
        h_out[...] = h
        if nxt is not None:
            xp_out[...] = jax.nn.relu(mm(h, wp_r[...]) + bp_r[...][None, :])

    res = pl.pallas_call(body, out_shape=outs)(*ins)
    return res if nxt is not None else (res[0], None)


_COS_TH = np.concatenate(
    [np.cos(np.arange(1, 128, dtype=np.float64) * np.pi / 256.0), [-2.0]]
).astype(np.float32).reshape(1, 128)


def kernel(vertices, segments, segment_edges, segment_mask, segment_edges_mask, params):
    B, NV, _ = vertices.shape
    NL = segments.shape[1]
    E = segment_edges.shape[1]
    N = B * NL
    seg = jnp.where(segment_mask[..., None], segments, 0)
    se0 = jnp.where(segment_edges_mask[..., None], segment_edges, 0)

    vpad = jnp.pad(vertices.reshape(B * NV, 3), ((0, 0), (0, 13)))
    boffV = (jnp.arange(B) * NV)[:, None]
    iA = (seg[..., 0] + boffV).reshape(-1).astype(jnp.int32)
    iB = (seg[..., 1] + boffV).reshape(-1).astype(jnp.int32)
    g1 = _sc_gather(vpad, jnp.concatenate([iA, iB]))
    vAB = jnp.concatenate([g1[:N, :3], g1[N:, :3]], axis=1)
    segrows = jnp.pad(vAB, ((0, 0), (0, 10)))

    boffL = (jnp.arange(B) * NL)[:, None]
    e0 = (se0[..., 0] + boffL).reshape(-1).astype(jnp.int32)
    e1 = (se0[..., 1] + boffL).reshape(-1).astype(jnp.int32)
    g2 = _sc_gather(segrows, jnp.concatenate([e0, e1]))
    ep0, ep1 = g2[:B * E], g2[B * E:]

    av = _tc_angle(ep0, ep1, params['angle_tab'], jnp.asarray(_COS_TH))
    nbins_a = 4224
    boffA = (jnp.arange(B) * (NL + 1))[:, None]
    b0 = (se0[..., 0] + boffA).reshape(-1).astype(jnp.int32)
    b1 = (se0[..., 1] + boffA).reshape(-1).astype(jnp.int32)
    ea = jnp.arange(B * E, dtype=jnp.int32)
    pa = _sc_segsum(av.astype(jnp.bfloat16), jnp.concatenate([ea, ea]),
                    jnp.concatenate([b0, b1]), nbins_a)
    ar = pa[:, :B * (NL + 1)].reshape(_NC, B, NL + 1, 32)[:, :, :NL].reshape(_NC, N, 32)

    p = params
    x, xp = _tc_embed(segrows, ar[0], ar[1], p['coor_tab'], p['length_tab'],
                      p['midp_tab'], p['dir_tab'], p['Win'], p['bin'],
                      p['sage'][0]['Wp'], p['sage'][0]['bp'])

    src = jnp.concatenate([e0, e1])
    dst = jnp.concatenate([e1, e0])
    cp = _sc_segsum(jnp.ones((_CH, 16), jnp.bfloat16), None, dst, N)
    h = x
    nlayers = len(p['sage'])
    for li in range(nlayers):
        pl_parts = _sc_segsum(xp.astype(jnp.bfloat16), src, dst, N)
        sp = p['sage'][li]
        ln = (p['ln_g'], p['ln_b']) if li == 0 else None
        nxt = None
        if li + 1 < nlayers:
            nxt = (p['sage'][li + 1]['Wp'], p['sage'][li + 1]['bp'])
        h, xp = _tc_sage(h, pl_parts[0], pl_parts[1], cp[0], cp[1],
                         sp['Wl'], sp['bl'], sp['Wr'], sp['br'], ln=ln, nxt=nxt)
    return h.reshape(B, NL, 384)
